# baseline (speedup 1.0000x reference)
import jax
import jax.numpy as jnp
from jax.experimental import pallas as pl
from jax.experimental.pallas import tpu as pltpu

_B = 1024
_C, _H, _W = 4, 64, 64
_CB = 16
_NCH = _B // _CB


def _fwd_kernel(t_sref, sac_sref, somac_sref, x_ref, n_ref, xt_ref, tg_ref):
    c = pl.program_id(0)
    rows = jax.lax.broadcasted_iota(jnp.int32, (_CB, 1, 1, 1), 0)
    sa = jnp.zeros((_CB, 1, 1, 1), jnp.float32)
    so = jnp.zeros((_CB, 1, 1, 1), jnp.float32)
    for i in range(_CB):
        ti = t_sref[c * _CB + i]
        sa = jnp.where(rows == i, sac_sref[ti], sa)
        so = jnp.where(rows == i, somac_sref[ti], so)
    x = x_ref[...]
    n = n_ref[...]
    xt_ref[...] = sa * x + so * n
    tg_ref[...] = sa * n - so * x


def kernel(x0, t, noise, sqrt_alphas_cumprod, sqrt_one_minus_alphas_cumprod):
    t32 = t.astype(jnp.int32)

    grid_spec = pltpu.PrefetchScalarGridSpec(
        num_scalar_prefetch=3,
        grid=(_NCH,),
        in_specs=[
            pl.BlockSpec((_CB, _C, _H, _W), lambda c, *_: (c, 0, 0, 0)),
            pl.BlockSpec((_CB, _C, _H, _W), lambda c, *_: (c, 0, 0, 0)),
        ],
        out_specs=[
            pl.BlockSpec((_CB, _C, _H, _W), lambda c, *_: (c, 0, 0, 0)),
            pl.BlockSpec((_CB, _C, _H, _W), lambda c, *_: (c, 0, 0, 0)),
        ],
    )
    xt, tgt = pl.pallas_call(
        _fwd_kernel,
        grid_spec=grid_spec,
        compiler_params=pltpu.CompilerParams(
            dimension_semantics=("parallel",),
        ),
        out_shape=[
            jax.ShapeDtypeStruct((_B, _C, _H, _W), jnp.float32),
            jax.ShapeDtypeStruct((_B, _C, _H, _W), jnp.float32),
        ],
    )(t32, sqrt_alphas_cumprod, sqrt_one_minus_alphas_cumprod, x0, noise)
    return xt, tgt


# speedup vs baseline: 1.8254x; 1.8254x over previous
import jax
import jax.numpy as jnp
from jax.experimental import pallas as pl
from jax.experimental.pallas import tpu as pltpu

_B = 1024
_C, _H, _W = 4, 64, 64
_COLS = _C * _H * _W
_CB = 64
_NCH = _B // _CB


def _fwd_kernel(t_sref, sac_sref, somac_sref, x_ref, n_ref, xt_ref, tg_ref):
    c = pl.program_id(0)
    rows = jax.lax.broadcasted_iota(jnp.int32, (_CB, 1), 0)
    sa = jnp.zeros((_CB, 1), jnp.float32)
    so = jnp.zeros((_CB, 1), jnp.float32)
    for i in range(_CB):
        ti = t_sref[c * _CB + i]
        sa = jnp.where(rows == i, sac_sref[ti], sa)
        so = jnp.where(rows == i, somac_sref[ti], so)
    x = x_ref[...]
    n = n_ref[...]
    xt_ref[...] = sa * x + so * n
    tg_ref[...] = sa * n - so * x


def kernel(x0, t, noise, sqrt_alphas_cumprod, sqrt_one_minus_alphas_cumprod):
    t32 = t.astype(jnp.int32)
    x2 = x0.reshape(_B, _COLS)
    n2 = noise.reshape(_B, _COLS)

    grid_spec = pltpu.PrefetchScalarGridSpec(
        num_scalar_prefetch=3,
        grid=(_NCH,),
        in_specs=[
            pl.BlockSpec((_CB, _COLS), lambda c, *_: (c, 0)),
            pl.BlockSpec((_CB, _COLS), lambda c, *_: (c, 0)),
        ],
        out_specs=[
            pl.BlockSpec((_CB, _COLS), lambda c, *_: (c, 0)),
            pl.BlockSpec((_CB, _COLS), lambda c, *_: (c, 0)),
        ],
    )
    xt, tgt = pl.pallas_call(
        _fwd_kernel,
        grid_spec=grid_spec,
        compiler_params=pltpu.CompilerParams(
            dimension_semantics=("parallel",),
        ),
        out_shape=[
            jax.ShapeDtypeStruct((_B, _COLS), jnp.float32),
            jax.ShapeDtypeStruct((_B, _COLS), jnp.float32),
        ],
    )(t32, sqrt_alphas_cumprod, sqrt_one_minus_alphas_cumprod, x2, n2)
    return xt.reshape(_B, _C, _H, _W), tgt.reshape(_B, _C, _H, _W)


# speedup vs baseline: 1.8309x; 1.0030x over previous
import jax
import jax.numpy as jnp
from jax.experimental import pallas as pl
from jax.experimental.pallas import tpu as pltpu

_B = 1024
_C, _H, _W = 4, 64, 64
_COLS = _C * _H * _W
_CB = 16
_NCH = _B // _CB
_D = 8


def _fwd_kernel(t_ref, sac_ref, somac_ref, x_hbm, n_hbm, xt_hbm, tg_hbm,
                xbuf, nbuf, xtbuf, tgbuf, sems):
    def in_start(i, slot):
        pltpu.make_async_copy(
            x_hbm.at[pl.ds(i * _CB, _CB), :], xbuf.at[slot], sems.at[0, slot]
        ).start()
        pltpu.make_async_copy(
            n_hbm.at[pl.ds(i * _CB, _CB), :], nbuf.at[slot], sems.at[1, slot]
        ).start()

    for d in range(_D):
        in_start(d, d)

    def step(i, carry):
        slot = jax.lax.rem(i, _D)
        pltpu.make_async_copy(
            x_hbm.at[pl.ds(i * _CB, _CB), :], xbuf.at[slot], sems.at[0, slot]
        ).wait()
        pltpu.make_async_copy(
            n_hbm.at[pl.ds(i * _CB, _CB), :], nbuf.at[slot], sems.at[1, slot]
        ).wait()

        rows = jax.lax.broadcasted_iota(jnp.int32, (_CB, 1), 0)
        sa = jnp.zeros((_CB, 1), jnp.float32)
        so = jnp.zeros((_CB, 1), jnp.float32)
        for k in range(_CB):
            ti = t_ref[i * _CB + k]
            sa = jnp.where(rows == k, sac_ref[ti], sa)
            so = jnp.where(rows == k, somac_ref[ti], so)

        @pl.when(i >= _D)
        def _():
            pltpu.make_async_copy(
                xtbuf.at[slot], xt_hbm.at[pl.ds((i - _D) * _CB, _CB), :],
                sems.at[2, slot],
            ).wait()
            pltpu.make_async_copy(
                tgbuf.at[slot], tg_hbm.at[pl.ds((i - _D) * _CB, _CB), :],
                sems.at[3, slot],
            ).wait()

        x = xbuf[slot]
        n = nbuf[slot]
        xtbuf[slot] = sa * x + so * n
        tgbuf[slot] = sa * n - so * x

        pltpu.make_async_copy(
            xtbuf.at[slot], xt_hbm.at[pl.ds(i * _CB, _CB), :], sems.at[2, slot]
        ).start()
        pltpu.make_async_copy(
            tgbuf.at[slot], tg_hbm.at[pl.ds(i * _CB, _CB), :], sems.at[3, slot]
        ).start()

        @pl.when(i + _D < _NCH)
        def _():
            in_start(i + _D, slot)

        return carry

    jax.lax.fori_loop(0, _NCH, step, 0)

    for d in range(_D):
        i = _NCH - _D + d
        pltpu.make_async_copy(
            xtbuf.at[d], xt_hbm.at[pl.ds(i * _CB, _CB), :], sems.at[2, d]
        ).wait()
        pltpu.make_async_copy(
            tgbuf.at[d], tg_hbm.at[pl.ds(i * _CB, _CB), :], sems.at[3, d]
        ).wait()


def kernel(x0, t, noise, sqrt_alphas_cumprod, sqrt_one_minus_alphas_cumprod):
    t32 = t.astype(jnp.int32)
    x2 = x0.reshape(_B, _COLS)
    n2 = noise.reshape(_B, _COLS)

    xt, tgt = pl.pallas_call(
        _fwd_kernel,
        in_specs=[
            pl.BlockSpec(memory_space=pltpu.SMEM),
            pl.BlockSpec(memory_space=pltpu.SMEM),
            pl.BlockSpec(memory_space=pltpu.SMEM),
            pl.BlockSpec(memory_space=pl.ANY),
            pl.BlockSpec(memory_space=pl.ANY),
        ],
        out_specs=[
            pl.BlockSpec(memory_space=pl.ANY),
            pl.BlockSpec(memory_space=pl.ANY),
        ],
        scratch_shapes=[
            pltpu.VMEM((_D, _CB, _COLS), jnp.float32),
            pltpu.VMEM((_D, _CB, _COLS), jnp.float32),
            pltpu.VMEM((_D, _CB, _COLS), jnp.float32),
            pltpu.VMEM((_D, _CB, _COLS), jnp.float32),
            pltpu.SemaphoreType.DMA((4, _D)),
        ],
        out_shape=[
            jax.ShapeDtypeStruct((_B, _COLS), jnp.float32),
            jax.ShapeDtypeStruct((_B, _COLS), jnp.float32),
        ],
    )(t32, sqrt_alphas_cumprod, sqrt_one_minus_alphas_cumprod, x2, n2)
    return xt.reshape(_B, _C, _H, _W), tgt.reshape(_B, _C, _H, _W)
